# Initial kernel scaffold; baseline (speedup 1.0000x reference)
import jax
import jax.numpy as jnp
from jax.experimental import pallas as pl


def kernel(logits, previous_tokens, current_step):
    raise NotImplementedError("write your pallas kernel here")


# speedup vs baseline: 2.8406x; 2.8406x over previous
import functools

import jax
import jax.numpy as jnp
import numpy as np
from jax import lax
from jax.experimental import pallas as pl
from jax.experimental.pallas import tpu as pltpu
from jax.experimental.pallas import tpu_sc as plsc

NC, NS, LANES = 2, 16, 16
NW = NC * NS

GT, PCT, EQ, HASH = 25, 14, 10, 11
NEG = -1e9

_DTBL = np.zeros(32, np.int32)
_DTBL[6] = 1; _DTBL[8] = 1
_DTBL[7] = -1; _DTBL[9] = -1


@functools.lru_cache(maxsize=None)
def _build(B, L, V):
    assert B % (NW * LANES) == 0 and L % 8 == 0 and V == 32
    rows_w = B // NW
    RB = min(128, rows_w)
    assert rows_w % RB == 0
    nblk = rows_w // RB

    mesh = plsc.VectorSubcoreMesh(
        core_axis_name="c", subcore_axis_name="s",
        num_cores=NC, num_subcores=NS)

    @functools.partial(
        pl.kernel,
        out_type=jax.ShapeDtypeStruct((B * V,), jnp.float32),
        mesh=mesh,
        compiler_params=pltpu.CompilerParams(needs_layout_passes=False),
        scratch_types=[
            pltpu.VMEM((RB * L,), jnp.int32),
            pltpu.VMEM((RB * V,), jnp.float32),
            pltpu.VMEM((32,), jnp.int32),
        ],
    )
    def sc_kernel(tok_hbm, log_hbm, dtbl_hbm, out_hbm, tok_v, out_v, dtbl_v):
        iota = lax.iota(jnp.int32, LANES)
        wid = lax.axis_index("s") * NC + lax.axis_index("c")
        pltpu.sync_copy(dtbl_hbm, dtbl_v)

        def group(rowv):
            full = lambda k: jnp.full((LANES,), k, jnp.int32)
            rbase = rowv * L
            lastv = plsc.load_gather(tok_v, [rbase + (L - 1)])
            t197 = plsc.load_gather(tok_v, [rbase + (L - 3)])
            t198 = plsc.load_gather(tok_v, [rbase + (L - 2)])

            def body(_, carry):
                idxv, c, prev, ring = carry
                for _u in range(8):
                    t = plsc.load_gather(tok_v, [idxv])
                    d = plsc.load_gather(dtbl_v, [t])
                    c = jnp.maximum(c + d, 0)
                    ring = jnp.where((prev == lastv) & (t == PCT), 1, ring)
                    prev = t
                    idxv = idxv + 1
                return idxv, c, prev, ring

            zero = jnp.zeros((LANES,), jnp.int32)
            _, c, prev, ring = lax.fori_loop(
                0, L // 8, body, (rbase, zero, full(-1), zero))

            bracket = c > 0
            ring_hit = (ring > 0) & (lastv >= 15) & (lastv <= 24)
            bond = (((t197 == EQ) | (t197 == HASH)).astype(jnp.int32)
                    + ((t198 == EQ) | (t198 == HASH)).astype(jnp.int32)
                    + ((lastv == EQ) | (lastv == HASH)).astype(jnp.int32))
            maxb = jnp.where(lastv == 0, 4,
                             jnp.where(lastv == 1, 2,
                                       jnp.where(lastv == 2, 3, 99)))
            val_hit = (lastv <= 2) & (bond >= maxb)

            neg = jnp.full((LANES,), NEG, jnp.float32)
            obase = rowv * V
            plsc.addupdate_scatter(out_v, [obase + GT], neg, mask=bracket)
            plsc.addupdate_scatter(out_v, [obase + lastv], neg, mask=ring_hit)
            plsc.addupdate_scatter(out_v, [obase + EQ], neg, mask=val_hit)
            plsc.addupdate_scatter(out_v, [obase + HASH], neg, mask=val_hit)

        for blk in range(nblk):
            base = wid * rows_w + blk * RB
            pltpu.sync_copy(tok_hbm.at[pl.ds(base * L, RB * L)], tok_v)
            pltpu.sync_copy(log_hbm.at[pl.ds(base * V, RB * V)], out_v)
            for g in range(RB // LANES):
                group(g * LANES + iota)
            pltpu.sync_copy(out_v, out_hbm.at[pl.ds(base * V, RB * V)])

    return sc_kernel


def kernel(logits, previous_tokens, current_step):
    del current_step
    tok = previous_tokens.astype(jnp.int32)
    logits = logits.astype(jnp.float32)
    B, L = tok.shape
    V = logits.shape[1]
    out = _build(B, L, V)(tok.reshape(-1), logits.reshape(-1), jnp.asarray(_DTBL))
    return out.reshape(B, V)
